# initial kernel scaffold (unmeasured)
import jax
import jax.numpy as jnp
from jax import lax
from jax.experimental import pallas as pl
from jax.experimental.pallas import tpu as pltpu

N_DEV = 32
B_LOC = 2
SQ = 128
SKV = 128
H_LOC = 4
DH = 64
D_MODEL = 512
HD_LOC = H_LOC * DH
ROWS = B_LOC * SQ


def _body(x_ref, chunk_ref, k_ref, v_ref, out_ref,
          comm_ref, ctx_ref, send_sems, recv_sems, credit_sem):
    my = lax.axis_index("i")
    left = lax.rem(my + N_DEV - 1, N_DEV)
    right = lax.rem(my + 1, N_DEV)

    barrier_sem = pltpu.get_barrier_semaphore()
    for nbr in (left, right):
        pl.semaphore_signal(barrier_sem, inc=1, device_id=(nbr,),
                            device_id_type=pl.DeviceIdType.MESH)
    pl.semaphore_wait(barrier_sem, 2)

    out_ref[:, :] = jnp.zeros((ROWS, D_MODEL), jnp.float32)
    comm_ref[0, :, :] = chunk_ref[:, :]

    def compute(jj, slot):
        chunk = comm_ref[pl.ds(slot, 1), :, :][0]
        wq_t = chunk[0:HD_LOC, :]
        wo = chunk[HD_LOC:2 * HD_LOC, :]
        qf = lax.dot_general(
            x_ref[:, :], wq_t, (((1,), (1,)), ((), ())),
            preferred_element_type=jnp.float32,
        ).astype(jnp.bfloat16)
        for b in range(B_LOC):
            for hh in range(H_LOC):
                q = qf[b * SQ:(b + 1) * SQ, hh * DH:(hh + 1) * DH]
                idx = jj * H_LOC + hh + b * (N_DEV * H_LOC)
                k = k_ref[pl.ds(idx, 1), :, :][0]
                s = lax.dot_general(
                    q, k, (((1,), (1,)), ((), ())),
                    preferred_element_type=jnp.float32,
                ) * 0.125
                m = jnp.max(s, axis=-1, keepdims=True)
                e = jnp.exp(s - m)
                w = (e / jnp.sum(e, axis=-1, keepdims=True)).astype(jnp.bfloat16)
                v = v_ref[pl.ds(idx, 1), :, :][0]
                c = lax.dot_general(
                    w, v, (((1,), (0,)), ((), ())),
                    preferred_element_type=jnp.float32,
                )
                ctx_ref[b * SQ:(b + 1) * SQ, hh * DH:(hh + 1) * DH] = (
                    c.astype(jnp.bfloat16))
        part = lax.dot_general(
            ctx_ref[:, :], wo, (((1,), (0,)), ((), ())),
            preferred_element_type=jnp.float32,
        )
        out_ref[:, :] = out_ref[:, :] + part

    def hop(h, carry):
        slot = lax.rem(h, 2)
        nxt = 1 - slot

        @pl.when(h > 0)
        def _():
            pl.semaphore_wait(credit_sem, 1)

        rdma = pltpu.make_async_remote_copy(
            src_ref=comm_ref.at[slot],
            dst_ref=comm_ref.at[nxt],
            send_sem=send_sems.at[slot],
            recv_sem=recv_sems.at[nxt],
            device_id=(right,),
            device_id_type=pl.DeviceIdType.MESH,
        )
        rdma.start()

        jj = lax.rem(my - h + 2 * N_DEV, N_DEV)
        compute(jj, slot)

        rdma.wait()

        @pl.when(h < N_DEV - 2)
        def _():
            pl.semaphore_signal(credit_sem, inc=1, device_id=(left,),
                                device_id_type=pl.DeviceIdType.MESH)
        return carry

    lax.fori_loop(0, N_DEV - 1, hop, 0)
    compute(lax.rem(my + 1, N_DEV), 1)


def kernel(x, Wq, K_ext, V_ext, Wo):
    my = lax.axis_index("i")
    xf = x.reshape(ROWS, D_MODEL).astype(jnp.bfloat16)
    chunk = jnp.concatenate([Wq.T, Wo], axis=0).astype(jnp.bfloat16)

    b0 = my * B_LOC
    k_loc = lax.dynamic_slice(K_ext, (b0, 0, 0, 0), (B_LOC, SKV, 128, DH))
    v_loc = lax.dynamic_slice(V_ext, (b0, 0, 0, 0), (B_LOC, SKV, 128, DH))
    k_t = k_loc.transpose(0, 2, 1, 3).reshape(B_LOC * 128, SKV, DH)
    v_t = v_loc.transpose(0, 2, 1, 3).reshape(B_LOC * 128, SKV, DH)
    k_t = k_t.astype(jnp.bfloat16)
    v_t = v_t.astype(jnp.bfloat16)

    out = pl.pallas_call(
        _body,
        out_shape=jax.ShapeDtypeStruct((ROWS, D_MODEL), jnp.float32),
        in_specs=[pl.BlockSpec(memory_space=pltpu.VMEM)] * 4,
        out_specs=pl.BlockSpec(memory_space=pltpu.VMEM),
        scratch_shapes=[
            pltpu.VMEM((2, 2 * HD_LOC, D_MODEL), jnp.bfloat16),
            pltpu.VMEM((ROWS, HD_LOC), jnp.bfloat16),
            pltpu.SemaphoreType.DMA((2,)),
            pltpu.SemaphoreType.DMA((2,)),
            pltpu.SemaphoreType.REGULAR,
        ],
        compiler_params=pltpu.CompilerParams(collective_id=0),
    )(xf, chunk, k_t, v_t)
    return out.reshape(B_LOC, SQ, D_MODEL)


# baseline (device time: 433773 ns/iter reference)
import jax
import jax.numpy as jnp
from jax import lax
from jax.experimental import pallas as pl
from jax.experimental.pallas import tpu as pltpu

N_DEV = 32
B_LOC = 2
SQ = 128
SKV = 128
H_LOC = 4
DH = 64
D_MODEL = 512
HD_LOC = H_LOC * DH
ROWS = B_LOC * SQ


def _body(x_ref, chunk_ref, k_ref, v_ref, out_ref,
          comm_ref, send_sems, recv_sems, credit_sem):
    my = lax.axis_index("i")
    left = lax.rem(my + N_DEV - 1, N_DEV)
    right = lax.rem(my + 1, N_DEV)

    barrier_sem = pltpu.get_barrier_semaphore()
    for nbr in (left, right):
        pl.semaphore_signal(barrier_sem, inc=1, device_id=(nbr,),
                            device_id_type=pl.DeviceIdType.MESH)
    pl.semaphore_wait(barrier_sem, 2)

    out_ref[:, :] = jnp.zeros((ROWS, D_MODEL), jnp.float32)
    comm_ref[0, :, :] = chunk_ref[:, :]

    def compute(jj, slot):

        def inner(p, carry):
            b = lax.div(p, H_LOC)
            hh = lax.rem(p, H_LOC)
            row0 = b * SQ
            xb = x_ref[pl.ds(row0, SQ), :]
            wq_p = comm_ref[pl.ds(slot, 1), pl.ds(hh * DH, DH), :][0]
            q = lax.dot_general(
                xb, wq_p, (((1,), (1,)), ((), ())),
                preferred_element_type=jnp.float32,
            ).astype(jnp.bfloat16)
            idx = b * (N_DEV * H_LOC) + jj * H_LOC + hh
            k = k_ref[pl.ds(idx, 1), :, :][0]
            s = lax.dot_general(
                q, k, (((1,), (1,)), ((), ())),
                preferred_element_type=jnp.float32,
            ) * 0.125
            m = jnp.max(s, axis=-1, keepdims=True)
            e = jnp.exp(s - m)
            w = (e / jnp.sum(e, axis=-1, keepdims=True)).astype(jnp.bfloat16)
            v = v_ref[pl.ds(idx, 1), :, :][0]
            c = lax.dot_general(
                w, v, (((1,), (0,)), ((), ())),
                preferred_element_type=jnp.float32,
            ).astype(jnp.bfloat16)
            wo_p = comm_ref[pl.ds(slot, 1), pl.ds(HD_LOC + hh * DH, DH), :][0]
            part = lax.dot_general(
                c, wo_p, (((1,), (0,)), ((), ())),
                preferred_element_type=jnp.float32,
            )
            out_ref[pl.ds(row0, SQ), :] = out_ref[pl.ds(row0, SQ), :] + part
            return carry

        lax.fori_loop(0, B_LOC * H_LOC, inner, 0)

    def hop(h, carry):
        slot = lax.rem(h, 2)
        nxt = 1 - slot

        @pl.when(jnp.logical_and(h > 0, h < N_DEV - 1))
        def _():
            pl.semaphore_wait(credit_sem, 1)

        rdma = pltpu.make_async_remote_copy(
            src_ref=comm_ref.at[slot],
            dst_ref=comm_ref.at[nxt],
            send_sem=send_sems.at[slot],
            recv_sem=recv_sems.at[nxt],
            device_id=(right,),
            device_id_type=pl.DeviceIdType.MESH,
        )

        @pl.when(h < N_DEV - 1)
        def _():
            rdma.start()

        jj = lax.rem(my - h + 2 * N_DEV, N_DEV)
        compute(jj, slot)

        @pl.when(h < N_DEV - 1)
        def _():
            rdma.wait()

        @pl.when(h < N_DEV - 2)
        def _():
            pl.semaphore_signal(credit_sem, inc=1, device_id=(left,),
                                device_id_type=pl.DeviceIdType.MESH)
        return carry

    lax.fori_loop(0, N_DEV, hop, 0)


def kernel(x, Wq, K_ext, V_ext, Wo):
    my = lax.axis_index("i")
    xf = x.reshape(ROWS, D_MODEL).astype(jnp.bfloat16)
    chunk = jnp.concatenate([Wq.T, Wo], axis=0).astype(jnp.bfloat16)

    b0 = my * B_LOC
    k_loc = lax.dynamic_slice(K_ext, (b0, 0, 0, 0), (B_LOC, SKV, 128, DH))
    v_loc = lax.dynamic_slice(V_ext, (b0, 0, 0, 0), (B_LOC, SKV, 128, DH))
    k_t = k_loc.transpose(0, 2, 1, 3).reshape(B_LOC * 128, SKV, DH)
    v_t = v_loc.transpose(0, 2, 1, 3).reshape(B_LOC * 128, SKV, DH)
    k_t = k_t.astype(jnp.bfloat16)
    v_t = v_t.astype(jnp.bfloat16)

    out = pl.pallas_call(
        _body,
        out_shape=jax.ShapeDtypeStruct((ROWS, D_MODEL), jnp.float32),
        in_specs=[pl.BlockSpec(memory_space=pltpu.VMEM)] * 4,
        out_specs=pl.BlockSpec(memory_space=pltpu.VMEM),
        scratch_shapes=[
            pltpu.VMEM((2, 2 * HD_LOC, D_MODEL), jnp.bfloat16),
            pltpu.SemaphoreType.DMA((2,)),
            pltpu.SemaphoreType.DMA((2,)),
            pltpu.SemaphoreType.REGULAR,
        ],
        compiler_params=pltpu.CompilerParams(collective_id=0),
    )(xf, chunk, k_t, v_t)
    return out.reshape(B_LOC, SQ, D_MODEL)


# device time: 175259 ns/iter; 2.4750x vs baseline; 2.4750x over previous
import jax
import jax.numpy as jnp
from jax import lax
from jax.experimental import pallas as pl
from jax.experimental.pallas import tpu as pltpu

N_DEV = 32
B_LOC = 2
SQ = 128
SKV = 128
H_LOC = 4
DH = 64
D_MODEL = 512
HD_LOC = H_LOC * DH
ROWS = B_LOC * SQ
N_HOP = N_DEV // 2

_H = [0, 1, 2, 5, 6, 7, 4, 3,
      11, 12, 15, 14, 13, 10, 9,
      17, 18, 21, 22, 23, 20, 19,
      27, 28, 31, 30, 29, 26, 25,
      24, 16, 8]
_H_INV = [0] * N_DEV
for _p, _i in enumerate(_H):
    _H_INV[_i] = _p


def _body(x_ref, chunk_ref, k_ref, v_ref, meta_ref, out_ref,
          comm_r, comm_l, s_ref, w_ref, ctx_ref,
          send_r, recv_r, send_l, recv_l, cred_r, cred_l):
    right = meta_ref[0]
    left = meta_ref[1]

    barrier_sem = pltpu.get_barrier_semaphore()
    for nbr in (left, right):
        pl.semaphore_signal(barrier_sem, inc=1, device_id=(nbr,),
                            device_id_type=pl.DeviceIdType.MESH)
    pl.semaphore_wait(barrier_sem, 2)

    out_ref[:, :] = jnp.zeros((ROWS, D_MODEL), jnp.float32)
    comm_r[0, :, :] = chunk_ref[:, :]
    comm_l[0, :, :] = chunk_ref[:, :]

    def compute(jj, comm, slot):
        for p in range(B_LOC * H_LOC):
            b, hh = divmod(p, H_LOC)
            xb = x_ref[b * SQ:(b + 1) * SQ, :]
            wq_p = comm[pl.ds(slot, 1), pl.ds(hh * DH, DH), :][0]
            q = lax.dot_general(
                xb, wq_p, (((1,), (1,)), ((), ())),
                preferred_element_type=jnp.float32,
            ).astype(jnp.bfloat16)
            idx = b * (N_DEV * H_LOC) + jj * H_LOC + hh
            k = k_ref[pl.ds(idx, 1), :, :][0]
            s_ref[p * SQ:(p + 1) * SQ, :] = lax.dot_general(
                q, k, (((1,), (1,)), ((), ())),
                preferred_element_type=jnp.float32,
            )
        sv = s_ref[:, :] * 0.125
        m = jnp.max(sv, axis=-1, keepdims=True)
        e = jnp.exp(sv - m)
        w_ref[:, :] = (e / jnp.sum(e, axis=-1, keepdims=True)
                       ).astype(jnp.bfloat16)
        for p in range(B_LOC * H_LOC):
            b, hh = divmod(p, H_LOC)
            idx = b * (N_DEV * H_LOC) + jj * H_LOC + hh
            v = v_ref[pl.ds(idx, 1), :, :][0]
            c = lax.dot_general(
                w_ref[p * SQ:(p + 1) * SQ, :], v,
                (((1,), (0,)), ((), ())),
                preferred_element_type=jnp.float32,
            ).astype(jnp.bfloat16)
            ctx_ref[b * SQ:(b + 1) * SQ, hh * DH:(hh + 1) * DH] = c
        wo = comm[pl.ds(slot, 1), pl.ds(HD_LOC, HD_LOC), :][0]
        part = lax.dot_general(
            ctx_ref[:, :], wo, (((1,), (0,)), ((), ())),
            preferred_element_type=jnp.float32,
        )
        out_ref[:, :] = out_ref[:, :] + part

    def hop(h, carry):
        slot = lax.rem(h, 2)
        nxt = 1 - slot

        @pl.when(h >= 1)
        def _():
            pl.semaphore_wait(cred_r, 1)

        @pl.when(jnp.logical_and(h >= 1, h <= N_HOP - 2))
        def _():
            pl.semaphore_wait(cred_l, 1)

        rdma_r = pltpu.make_async_remote_copy(
            src_ref=comm_r.at[slot], dst_ref=comm_r.at[nxt],
            send_sem=send_r.at[slot], recv_sem=recv_r.at[nxt],
            device_id=(right,), device_id_type=pl.DeviceIdType.MESH,
        )
        rdma_l = pltpu.make_async_remote_copy(
            src_ref=comm_l.at[slot], dst_ref=comm_l.at[nxt],
            send_sem=send_l.at[slot], recv_sem=recv_l.at[nxt],
            device_id=(left,), device_id_type=pl.DeviceIdType.MESH,
        )
        rdma_r.start()

        @pl.when(h <= N_HOP - 2)
        def _():
            rdma_l.start()

        compute(meta_ref[2 + h], comm_r, slot)

        @pl.when(h >= 1)
        def _():
            compute(meta_ref[2 + N_HOP + 1 + h], comm_l, slot)

        rdma_r.wait()

        @pl.when(h <= N_HOP - 2)
        def _():
            rdma_l.wait()

        @pl.when(h <= N_HOP - 2)
        def _():
            pl.semaphore_signal(cred_r, inc=1, device_id=(left,),
                                device_id_type=pl.DeviceIdType.MESH)

        @pl.when(h <= N_HOP - 3)
        def _():
            pl.semaphore_signal(cred_l, inc=1, device_id=(right,),
                                device_id_type=pl.DeviceIdType.MESH)
        return carry

    lax.fori_loop(0, N_HOP, hop, 0)
    compute(meta_ref[2 + N_HOP], comm_r, 0)


def kernel(x, Wq, K_ext, V_ext, Wo):
    my = lax.axis_index("i")
    xf = x.reshape(ROWS, D_MODEL).astype(jnp.bfloat16)
    chunk = jnp.concatenate([Wq.T, Wo], axis=0).astype(jnp.bfloat16)

    b0 = my * B_LOC
    k_loc = lax.dynamic_slice(K_ext, (b0, 0, 0, 0), (B_LOC, SKV, 128, DH))
    v_loc = lax.dynamic_slice(V_ext, (b0, 0, 0, 0), (B_LOC, SKV, 128, DH))
    k_t = k_loc.transpose(0, 2, 1, 3).reshape(B_LOC * 128, SKV, DH)
    v_t = v_loc.transpose(0, 2, 1, 3).reshape(B_LOC * 128, SKV, DH)
    k_t = k_t.astype(jnp.bfloat16)
    v_t = v_t.astype(jnp.bfloat16)

    hj = jnp.array(_H, jnp.int32)
    p = jnp.array(_H_INV, jnp.int32)[my]
    right = hj[jnp.remainder(p + 1, N_DEV)]
    left = hj[jnp.remainder(p - 1, N_DEV)]
    orr = hj[jnp.remainder(p - jnp.arange(N_HOP + 1), N_DEV)]
    orl = hj[jnp.remainder(p + jnp.arange(N_HOP), N_DEV)]
    meta = jnp.concatenate(
        [right[None], left[None], orr, orl]).astype(jnp.int32)

    out = pl.pallas_call(
        _body,
        out_shape=jax.ShapeDtypeStruct((ROWS, D_MODEL), jnp.float32),
        in_specs=[pl.BlockSpec(memory_space=pltpu.VMEM)] * 4
        + [pl.BlockSpec(memory_space=pltpu.SMEM)],
        out_specs=pl.BlockSpec(memory_space=pltpu.VMEM),
        scratch_shapes=[
            pltpu.VMEM((2, 2 * HD_LOC, D_MODEL), jnp.bfloat16),
            pltpu.VMEM((2, 2 * HD_LOC, D_MODEL), jnp.bfloat16),
            pltpu.VMEM((B_LOC * H_LOC * SQ, SKV), jnp.float32),
            pltpu.VMEM((B_LOC * H_LOC * SQ, SKV), jnp.bfloat16),
            pltpu.VMEM((ROWS, HD_LOC), jnp.bfloat16),
            pltpu.SemaphoreType.DMA((2,)),
            pltpu.SemaphoreType.DMA((2,)),
            pltpu.SemaphoreType.DMA((2,)),
            pltpu.SemaphoreType.DMA((2,)),
            pltpu.SemaphoreType.REGULAR,
            pltpu.SemaphoreType.REGULAR,
        ],
        compiler_params=pltpu.CompilerParams(collective_id=0),
    )(xf, chunk, k_t, v_t, meta)
    return out.reshape(B_LOC, SQ, D_MODEL)


# device time: 164508 ns/iter; 2.6368x vs baseline; 1.0654x over previous
import jax
import jax.numpy as jnp
from jax import lax
from jax.experimental import pallas as pl
from jax.experimental.pallas import tpu as pltpu

N_DEV = 32
B_LOC = 2
SQ = 128
SKV = 128
H_LOC = 4
DH = 64
D_MODEL = 512
HD_LOC = H_LOC * DH
ROWS = B_LOC * SQ
N_HOP = N_DEV // 2

_H = [0, 1, 2, 5, 6, 7, 4, 3,
      11, 12, 15, 14, 13, 10, 9,
      17, 18, 21, 22, 23, 20, 19,
      27, 28, 31, 30, 29, 26, 25,
      24, 16, 8]
_H_INV = [0] * N_DEV
for _p, _i in enumerate(_H):
    _H_INV[_i] = _p


def _body(x_ref, chunk_ref, k_ref, v_ref, meta_ref, out_ref,
          comm_r, comm_l, s_ref, w_ref, ctx_ref,
          send_r, recv_r, send_l, recv_l, cred_r, cred_l):
    right = meta_ref[0]
    left = meta_ref[1]

    barrier_sem = pltpu.get_barrier_semaphore()
    for nbr in (left, right):
        pl.semaphore_signal(barrier_sem, inc=1, device_id=(nbr,),
                            device_id_type=pl.DeviceIdType.MESH)
    pl.semaphore_wait(barrier_sem, 2)

    out_ref[:, :] = jnp.zeros((ROWS, D_MODEL), jnp.float32)
    comm_r[0, :, :] = chunk_ref[:, :]
    comm_l[0, :, :] = chunk_ref[:, :]

    def compute(jj, comm, slot):
        for p in range(B_LOC * H_LOC):
            b, hh = divmod(p, H_LOC)
            xb = x_ref[b * SQ:(b + 1) * SQ, :]
            wq_p = comm[pl.ds(slot, 1), pl.ds(hh * DH, DH), :][0]
            q = lax.dot_general(
                xb, wq_p, (((1,), (1,)), ((), ())),
                preferred_element_type=jnp.float32,
            ).astype(jnp.bfloat16)
            idx = b * (N_DEV * H_LOC) + jj * H_LOC + hh
            k = k_ref[pl.ds(idx, 1), :, :][0]
            s_ref[p * SQ:(p + 1) * SQ, :] = lax.dot_general(
                q, k, (((1,), (1,)), ((), ())),
                preferred_element_type=jnp.float32,
            )
        sv = s_ref[:, :] * 0.125
        m = jnp.max(sv, axis=-1, keepdims=True)
        e = jnp.exp(sv - m)
        w_ref[:, :] = (e / jnp.sum(e, axis=-1, keepdims=True)
                       ).astype(jnp.bfloat16)
        for p in range(B_LOC * H_LOC):
            b, hh = divmod(p, H_LOC)
            idx = b * (N_DEV * H_LOC) + jj * H_LOC + hh
            v = v_ref[pl.ds(idx, 1), :, :][0]
            c = lax.dot_general(
                w_ref[p * SQ:(p + 1) * SQ, :], v,
                (((1,), (0,)), ((), ())),
                preferred_element_type=jnp.float32,
            ).astype(jnp.bfloat16)
            ctx_ref[b * SQ:(b + 1) * SQ, hh * DH:(hh + 1) * DH] = c
        wo = comm[pl.ds(slot, 1), pl.ds(HD_LOC, HD_LOC), :][0]
        part = lax.dot_general(
            ctx_ref[:, :], wo, (((1,), (0,)), ((), ())),
            preferred_element_type=jnp.float32,
        )
        out_ref[:, :] = out_ref[:, :] + part

    def hop(h, carry):
        slot = lax.rem(h, 4)
        nxt = lax.rem(h + 1, 4)

        @pl.when(h >= 3)
        def _():
            pl.semaphore_wait(cred_r, 1)

        @pl.when(jnp.logical_and(h >= 3, h <= N_HOP - 2))
        def _():
            pl.semaphore_wait(cred_l, 1)

        rdma_r = pltpu.make_async_remote_copy(
            src_ref=comm_r.at[slot], dst_ref=comm_r.at[nxt],
            send_sem=send_r.at[slot], recv_sem=recv_r.at[nxt],
            device_id=(right,), device_id_type=pl.DeviceIdType.MESH,
        )
        rdma_l = pltpu.make_async_remote_copy(
            src_ref=comm_l.at[slot], dst_ref=comm_l.at[nxt],
            send_sem=send_l.at[slot], recv_sem=recv_l.at[nxt],
            device_id=(left,), device_id_type=pl.DeviceIdType.MESH,
        )
        rdma_r.start()

        @pl.when(h <= N_HOP - 2)
        def _():
            rdma_l.start()

        compute(meta_ref[2 + h], comm_r, slot)

        @pl.when(h >= 1)
        def _():
            compute(meta_ref[2 + N_HOP + 1 + h], comm_l, slot)

        rdma_r.wait()

        @pl.when(h <= N_HOP - 2)
        def _():
            rdma_l.wait()

        @pl.when(h <= N_HOP - 4)
        def _():
            pl.semaphore_signal(cred_r, inc=1, device_id=(left,),
                                device_id_type=pl.DeviceIdType.MESH)

        @pl.when(h <= N_HOP - 5)
        def _():
            pl.semaphore_signal(cred_l, inc=1, device_id=(right,),
                                device_id_type=pl.DeviceIdType.MESH)
        return carry

    lax.fori_loop(0, N_HOP, hop, 0)
    compute(meta_ref[2 + N_HOP], comm_r, 0)


def kernel(x, Wq, K_ext, V_ext, Wo):
    my = lax.axis_index("i")
    xf = x.reshape(ROWS, D_MODEL).astype(jnp.bfloat16)
    chunk = jnp.concatenate([Wq.T, Wo], axis=0).astype(jnp.bfloat16)

    b0 = my * B_LOC
    k_loc = lax.dynamic_slice(K_ext, (b0, 0, 0, 0), (B_LOC, SKV, 128, DH))
    v_loc = lax.dynamic_slice(V_ext, (b0, 0, 0, 0), (B_LOC, SKV, 128, DH))
    k_t = k_loc.transpose(0, 2, 1, 3).reshape(B_LOC * 128, SKV, DH)
    v_t = v_loc.transpose(0, 2, 1, 3).reshape(B_LOC * 128, SKV, DH)
    k_t = k_t.astype(jnp.bfloat16)
    v_t = v_t.astype(jnp.bfloat16)

    hj = jnp.array(_H, jnp.int32)
    p = jnp.array(_H_INV, jnp.int32)[my]
    right = hj[jnp.remainder(p + 1, N_DEV)]
    left = hj[jnp.remainder(p - 1, N_DEV)]
    orr = hj[jnp.remainder(p - jnp.arange(N_HOP + 1), N_DEV)]
    orl = hj[jnp.remainder(p + jnp.arange(N_HOP), N_DEV)]
    meta = jnp.concatenate(
        [right[None], left[None], orr, orl]).astype(jnp.int32)

    out = pl.pallas_call(
        _body,
        out_shape=jax.ShapeDtypeStruct((ROWS, D_MODEL), jnp.float32),
        in_specs=[pl.BlockSpec(memory_space=pltpu.VMEM)] * 4
        + [pl.BlockSpec(memory_space=pltpu.SMEM)],
        out_specs=pl.BlockSpec(memory_space=pltpu.VMEM),
        scratch_shapes=[
            pltpu.VMEM((4, 2 * HD_LOC, D_MODEL), jnp.bfloat16),
            pltpu.VMEM((4, 2 * HD_LOC, D_MODEL), jnp.bfloat16),
            pltpu.VMEM((B_LOC * H_LOC * SQ, SKV), jnp.float32),
            pltpu.VMEM((B_LOC * H_LOC * SQ, SKV), jnp.bfloat16),
            pltpu.VMEM((ROWS, HD_LOC), jnp.bfloat16),
            pltpu.SemaphoreType.DMA((4,)),
            pltpu.SemaphoreType.DMA((4,)),
            pltpu.SemaphoreType.DMA((4,)),
            pltpu.SemaphoreType.DMA((4,)),
            pltpu.SemaphoreType.REGULAR,
            pltpu.SemaphoreType.REGULAR,
        ],
        compiler_params=pltpu.CompilerParams(collective_id=0),
    )(xf, chunk, k_t, v_t, meta)
    return out.reshape(B_LOC, SQ, D_MODEL)


# device time: 163516 ns/iter; 2.6528x vs baseline; 1.0061x over previous
import jax
import jax.numpy as jnp
from jax import lax
from jax.experimental import pallas as pl
from jax.experimental.pallas import tpu as pltpu

N_DEV = 32
B_LOC = 2
SQ = 128
SKV = 128
H_LOC = 4
DH = 64
D_MODEL = 512
HD_LOC = H_LOC * DH
ROWS = B_LOC * SQ
N_HOP = N_DEV // 2

_H = [0, 1, 2, 5, 6, 7, 4, 3,
      11, 12, 15, 14, 13, 10, 9,
      17, 18, 21, 22, 23, 20, 19,
      27, 28, 31, 30, 29, 26, 25,
      24, 16, 8]
_H_INV = [0] * N_DEV
for _p, _i in enumerate(_H):
    _H_INV[_i] = _p


def _body(x_ref, chunk_ref, k_ref, v_ref, meta_ref, out_ref,
          comm_r, comm_l, s_ref, w_ref, ctx_ref,
          send_r, recv_r, send_l, recv_l, cred_r, cred_l):
    right = meta_ref[0]
    left = meta_ref[1]

    barrier_sem = pltpu.get_barrier_semaphore()
    for nbr in (left, right):
        pl.semaphore_signal(barrier_sem, inc=1, device_id=(nbr,),
                            device_id_type=pl.DeviceIdType.MESH)
    pl.semaphore_wait(barrier_sem, 2)

    out_ref[:, :] = jnp.zeros((ROWS, D_MODEL), jnp.float32)
    comm_r[0, :, :] = chunk_ref[:, :]
    comm_l[0, :, :] = chunk_ref[:, :]

    def compute(jj, comm, slot):
        wq_t = comm[pl.ds(slot, 1), pl.ds(0, HD_LOC), :][0]
        qf = (lax.dot_general(
            x_ref[:, :], wq_t, (((1,), (1,)), ((), ())),
            preferred_element_type=jnp.float32,
        ) * 0.125).astype(jnp.bfloat16)
        for p in range(B_LOC * H_LOC):
            b, hh = divmod(p, H_LOC)
            q = qf[b * SQ:(b + 1) * SQ, hh * DH:(hh + 1) * DH]
            idx = b * (N_DEV * H_LOC) + jj * H_LOC + hh
            k = k_ref[pl.ds(idx, 1), :, :][0]
            s_ref[p * SQ:(p + 1) * SQ, :] = lax.dot_general(
                q, k, (((1,), (1,)), ((), ())),
                preferred_element_type=jnp.float32,
            )
        sv = s_ref[:, :]
        m = jnp.max(sv, axis=-1, keepdims=True)
        e = jnp.exp(sv - m)
        w_ref[:, :] = (e / jnp.sum(e, axis=-1, keepdims=True)
                       ).astype(jnp.bfloat16)
        for p in range(B_LOC * H_LOC):
            b, hh = divmod(p, H_LOC)
            idx = b * (N_DEV * H_LOC) + jj * H_LOC + hh
            v = v_ref[pl.ds(idx, 1), :, :][0]
            c = lax.dot_general(
                w_ref[p * SQ:(p + 1) * SQ, :], v,
                (((1,), (0,)), ((), ())),
                preferred_element_type=jnp.float32,
            ).astype(jnp.bfloat16)
            ctx_ref[b * SQ:(b + 1) * SQ, hh * DH:(hh + 1) * DH] = c
        wo = comm[pl.ds(slot, 1), pl.ds(HD_LOC, HD_LOC), :][0]
        part = lax.dot_general(
            ctx_ref[:, :], wo, (((1,), (0,)), ((), ())),
            preferred_element_type=jnp.float32,
        )
        out_ref[:, :] = out_ref[:, :] + part

    def hop(h, carry):
        slot = lax.rem(h, 4)
        nxt = lax.rem(h + 1, 4)

        @pl.when(h >= 3)
        def _():
            pl.semaphore_wait(cred_r, 1)

        @pl.when(jnp.logical_and(h >= 3, h <= N_HOP - 2))
        def _():
            pl.semaphore_wait(cred_l, 1)

        rdma_r = pltpu.make_async_remote_copy(
            src_ref=comm_r.at[slot], dst_ref=comm_r.at[nxt],
            send_sem=send_r.at[slot], recv_sem=recv_r.at[nxt],
            device_id=(right,), device_id_type=pl.DeviceIdType.MESH,
        )
        rdma_l = pltpu.make_async_remote_copy(
            src_ref=comm_l.at[slot], dst_ref=comm_l.at[nxt],
            send_sem=send_l.at[slot], recv_sem=recv_l.at[nxt],
            device_id=(left,), device_id_type=pl.DeviceIdType.MESH,
        )
        rdma_r.start()

        @pl.when(h <= N_HOP - 2)
        def _():
            rdma_l.start()

        compute(meta_ref[2 + h], comm_r, slot)

        @pl.when(h >= 1)
        def _():
            compute(meta_ref[2 + N_HOP + 1 + h], comm_l, slot)

        rdma_r.wait()

        @pl.when(h <= N_HOP - 2)
        def _():
            rdma_l.wait()

        @pl.when(h <= N_HOP - 4)
        def _():
            pl.semaphore_signal(cred_r, inc=1, device_id=(left,),
                                device_id_type=pl.DeviceIdType.MESH)

        @pl.when(h <= N_HOP - 5)
        def _():
            pl.semaphore_signal(cred_l, inc=1, device_id=(right,),
                                device_id_type=pl.DeviceIdType.MESH)
        return carry

    lax.fori_loop(0, N_HOP, hop, 0)
    compute(meta_ref[2 + N_HOP], comm_r, 0)


def kernel(x, Wq, K_ext, V_ext, Wo):
    my = lax.axis_index("i")
    xf = x.reshape(ROWS, D_MODEL).astype(jnp.bfloat16)
    chunk = jnp.concatenate([Wq.T, Wo], axis=0).astype(jnp.bfloat16)

    b0 = my * B_LOC
    k_loc = lax.dynamic_slice(K_ext, (b0, 0, 0, 0), (B_LOC, SKV, 128, DH))
    v_loc = lax.dynamic_slice(V_ext, (b0, 0, 0, 0), (B_LOC, SKV, 128, DH))
    k_t = k_loc.transpose(0, 2, 1, 3).reshape(B_LOC * 128, SKV, DH)
    v_t = v_loc.transpose(0, 2, 1, 3).reshape(B_LOC * 128, SKV, DH)
    k_t = k_t.astype(jnp.bfloat16)
    v_t = v_t.astype(jnp.bfloat16)

    hj = jnp.array(_H, jnp.int32)
    p = jnp.array(_H_INV, jnp.int32)[my]
    right = hj[jnp.remainder(p + 1, N_DEV)]
    left = hj[jnp.remainder(p - 1, N_DEV)]
    orr = hj[jnp.remainder(p - jnp.arange(N_HOP + 1), N_DEV)]
    orl = hj[jnp.remainder(p + jnp.arange(N_HOP), N_DEV)]
    meta = jnp.concatenate(
        [right[None], left[None], orr, orl]).astype(jnp.int32)

    out = pl.pallas_call(
        _body,
        out_shape=jax.ShapeDtypeStruct((ROWS, D_MODEL), jnp.float32),
        in_specs=[pl.BlockSpec(memory_space=pltpu.VMEM)] * 4
        + [pl.BlockSpec(memory_space=pltpu.SMEM)],
        out_specs=pl.BlockSpec(memory_space=pltpu.VMEM),
        scratch_shapes=[
            pltpu.VMEM((4, 2 * HD_LOC, D_MODEL), jnp.bfloat16),
            pltpu.VMEM((4, 2 * HD_LOC, D_MODEL), jnp.bfloat16),
            pltpu.VMEM((B_LOC * H_LOC * SQ, SKV), jnp.float32),
            pltpu.VMEM((B_LOC * H_LOC * SQ, SKV), jnp.bfloat16),
            pltpu.VMEM((ROWS, HD_LOC), jnp.bfloat16),
            pltpu.SemaphoreType.DMA((4,)),
            pltpu.SemaphoreType.DMA((4,)),
            pltpu.SemaphoreType.DMA((4,)),
            pltpu.SemaphoreType.DMA((4,)),
            pltpu.SemaphoreType.REGULAR,
            pltpu.SemaphoreType.REGULAR,
        ],
        compiler_params=pltpu.CompilerParams(collective_id=0),
    )(xf, chunk, k_t, v_t, meta)
    return out.reshape(B_LOC, SQ, D_MODEL)


# device time: 162210 ns/iter; 2.6741x vs baseline; 1.0081x over previous
import jax
import jax.numpy as jnp
from jax import lax
from jax.experimental import pallas as pl
from jax.experimental.pallas import tpu as pltpu

N_DEV = 32
B_LOC = 2
SQ = 128
SKV = 128
H_LOC = 4
DH = 64
D_MODEL = 512
HD_LOC = H_LOC * DH
ROWS = B_LOC * SQ
N_HOP = N_DEV // 2

_H = [0, 1, 2, 5, 6, 7, 4, 3,
      11, 12, 15, 14, 13, 10, 9,
      17, 18, 21, 22, 23, 20, 19,
      27, 28, 31, 30, 29, 26, 25,
      24, 16, 8]
_H_INV = [0] * N_DEV
for _p, _i in enumerate(_H):
    _H_INV[_i] = _p


def _body(x_ref, chunk_ref, k_ref, v_ref, meta_ref, out_ref,
          comm_r, comm_l, s_ref, w_ref, ctx_ref,
          send_ra, recv_ra, send_rb, recv_rb,
          send_la, recv_la, send_lb, recv_lb, cred_r, cred_l):
    right = meta_ref[0]
    left = meta_ref[1]

    barrier_sem = pltpu.get_barrier_semaphore()
    for nbr in (left, right):
        pl.semaphore_signal(barrier_sem, inc=1, device_id=(nbr,),
                            device_id_type=pl.DeviceIdType.MESH)
    pl.semaphore_wait(barrier_sem, 2)

    out_ref[:, :] = jnp.zeros((ROWS, D_MODEL), jnp.float32)
    comm_r[0, :, :] = chunk_ref[:, :]
    comm_l[0, :, :] = chunk_ref[:, :]

    def compute1(jj, comm, slot):
        wq_t = comm[pl.ds(slot, 1), pl.ds(0, HD_LOC), :][0]
        qf = (lax.dot_general(
            x_ref[:, :], wq_t, (((1,), (1,)), ((), ())),
            preferred_element_type=jnp.float32,
        ) * 0.125).astype(jnp.bfloat16)
        for p in range(B_LOC * H_LOC):
            b, hh = divmod(p, H_LOC)
            q = qf[b * SQ:(b + 1) * SQ, hh * DH:(hh + 1) * DH]
            idx = b * (N_DEV * H_LOC) + jj * H_LOC + hh
            k = k_ref[pl.ds(idx, 1), :, :][0]
            s_ref[p * SQ:(p + 1) * SQ, :] = lax.dot_general(
                q, k, (((1,), (1,)), ((), ())),
                preferred_element_type=jnp.float32,
            )
        sv = s_ref[:, :]
        m = jnp.max(sv, axis=-1, keepdims=True)
        e = jnp.exp(sv - m)
        w_ref[:, :] = (e / jnp.sum(e, axis=-1, keepdims=True)
                       ).astype(jnp.bfloat16)
        for p in range(B_LOC * H_LOC):
            b, hh = divmod(p, H_LOC)
            idx = b * (N_DEV * H_LOC) + jj * H_LOC + hh
            v = v_ref[pl.ds(idx, 1), :, :][0]
            c = lax.dot_general(
                w_ref[p * SQ:(p + 1) * SQ, :], v,
                (((1,), (0,)), ((), ())),
                preferred_element_type=jnp.float32,
            ).astype(jnp.bfloat16)
            ctx_ref[b * SQ:(b + 1) * SQ, hh * DH:(hh + 1) * DH] = c

    def compute2(comm, slot):
        wo = comm[pl.ds(slot, 1), pl.ds(HD_LOC, HD_LOC), :][0]
        part = lax.dot_general(
            ctx_ref[:, :], wo, (((1,), (0,)), ((), ())),
            preferred_element_type=jnp.float32,
        )
        out_ref[:, :] = out_ref[:, :] + part

    def half(buf, src_slot, dst_slot, row0, s_sem, r_sem, r_idx, dev):
        return pltpu.make_async_remote_copy(
            src_ref=buf.at[src_slot, pl.ds(row0, HD_LOC)],
            dst_ref=buf.at[dst_slot, pl.ds(row0, HD_LOC)],
            send_sem=s_sem.at[src_slot], recv_sem=r_sem.at[r_idx],
            device_id=(dev,), device_id_type=pl.DeviceIdType.MESH,
        )

    def hop(h, carry):
        slot = lax.rem(h, 4)
        nxt = lax.rem(h + 1, 4)

        @pl.when(h >= 3)
        def _():
            pl.semaphore_wait(cred_r, 1)

        @pl.when(jnp.logical_and(h >= 3, h <= N_HOP - 2))
        def _():
            pl.semaphore_wait(cred_l, 1)

        @pl.when(h >= 1)
        def _():
            half(comm_r, slot, nxt, 0, send_ra, recv_ra, slot,
                 right).wait_recv()
        fwd_ra = half(comm_r, slot, nxt, 0, send_ra, recv_ra, nxt, right)
        fwd_ra.start()
        compute1(meta_ref[2 + h], comm_r, slot)

        @pl.when(h >= 1)
        def _():
            half(comm_r, slot, nxt, HD_LOC, send_rb, recv_rb, slot,
                 right).wait_recv()
        fwd_rb = half(comm_r, slot, nxt, HD_LOC, send_rb, recv_rb, nxt,
                      right)
        fwd_rb.start()
        compute2(comm_r, slot)

        @pl.when(h >= 1)
        def _():
            half(comm_l, slot, nxt, 0, send_la, recv_la, slot,
                 left).wait_recv()
        fwd_la = half(comm_l, slot, nxt, 0, send_la, recv_la, nxt, left)

        @pl.when(h <= N_HOP - 2)
        def _():
            fwd_la.start()

        @pl.when(h >= 1)
        def _():
            compute1(meta_ref[2 + N_HOP + 1 + h], comm_l, slot)
            half(comm_l, slot, nxt, HD_LOC, send_lb, recv_lb, slot,
                 left).wait_recv()

        fwd_lb = half(comm_l, slot, nxt, HD_LOC, send_lb, recv_lb, nxt,
                      left)

        @pl.when(h <= N_HOP - 2)
        def _():
            fwd_lb.start()

        @pl.when(h >= 1)
        def _():
            compute2(comm_l, slot)

        fwd_ra.wait_send()
        fwd_rb.wait_send()

        @pl.when(h <= N_HOP - 2)
        def _():
            fwd_la.wait_send()
            fwd_lb.wait_send()

        @pl.when(h <= N_HOP - 4)
        def _():
            pl.semaphore_signal(cred_r, inc=1, device_id=(left,),
                                device_id_type=pl.DeviceIdType.MESH)

        @pl.when(h <= N_HOP - 5)
        def _():
            pl.semaphore_signal(cred_l, inc=1, device_id=(right,),
                                device_id_type=pl.DeviceIdType.MESH)
        return carry

    lax.fori_loop(0, N_HOP, hop, 0)
    half(comm_r, 0, 0, 0, send_ra, recv_ra, 0, right).wait_recv()
    compute1(meta_ref[2 + N_HOP], comm_r, 0)
    half(comm_r, 0, 0, HD_LOC, send_rb, recv_rb, 0, right).wait_recv()
    compute2(comm_r, 0)


def kernel(x, Wq, K_ext, V_ext, Wo):
    my = lax.axis_index("i")
    xf = x.reshape(ROWS, D_MODEL).astype(jnp.bfloat16)
    chunk = jnp.concatenate([Wq.T, Wo], axis=0).astype(jnp.bfloat16)

    b0 = my * B_LOC
    k_loc = lax.dynamic_slice(K_ext, (b0, 0, 0, 0), (B_LOC, SKV, 128, DH))
    v_loc = lax.dynamic_slice(V_ext, (b0, 0, 0, 0), (B_LOC, SKV, 128, DH))
    k_t = k_loc.transpose(0, 2, 1, 3).reshape(B_LOC * 128, SKV, DH)
    v_t = v_loc.transpose(0, 2, 1, 3).reshape(B_LOC * 128, SKV, DH)
    k_t = k_t.astype(jnp.bfloat16)
    v_t = v_t.astype(jnp.bfloat16)

    hj = jnp.array(_H, jnp.int32)
    p = jnp.array(_H_INV, jnp.int32)[my]
    right = hj[jnp.remainder(p + 1, N_DEV)]
    left = hj[jnp.remainder(p - 1, N_DEV)]
    orr = hj[jnp.remainder(p - jnp.arange(N_HOP + 1), N_DEV)]
    orl = hj[jnp.remainder(p + jnp.arange(N_HOP), N_DEV)]
    meta = jnp.concatenate(
        [right[None], left[None], orr, orl]).astype(jnp.int32)

    out = pl.pallas_call(
        _body,
        out_shape=jax.ShapeDtypeStruct((ROWS, D_MODEL), jnp.float32),
        in_specs=[pl.BlockSpec(memory_space=pltpu.VMEM)] * 4
        + [pl.BlockSpec(memory_space=pltpu.SMEM)],
        out_specs=pl.BlockSpec(memory_space=pltpu.VMEM),
        scratch_shapes=[
            pltpu.VMEM((4, 2 * HD_LOC, D_MODEL), jnp.bfloat16),
            pltpu.VMEM((4, 2 * HD_LOC, D_MODEL), jnp.bfloat16),
            pltpu.VMEM((B_LOC * H_LOC * SQ, SKV), jnp.float32),
            pltpu.VMEM((B_LOC * H_LOC * SQ, SKV), jnp.bfloat16),
            pltpu.VMEM((ROWS, HD_LOC), jnp.bfloat16),
            pltpu.SemaphoreType.DMA((4,)),
            pltpu.SemaphoreType.DMA((4,)),
            pltpu.SemaphoreType.DMA((4,)),
            pltpu.SemaphoreType.DMA((4,)),
            pltpu.SemaphoreType.DMA((4,)),
            pltpu.SemaphoreType.DMA((4,)),
            pltpu.SemaphoreType.DMA((4,)),
            pltpu.SemaphoreType.DMA((4,)),
            pltpu.SemaphoreType.REGULAR,
            pltpu.SemaphoreType.REGULAR,
        ],
        compiler_params=pltpu.CompilerParams(collective_id=0),
    )(xf, chunk, k_t, v_t, meta)
    return out.reshape(B_LOC, SQ, D_MODEL)


# device time: 153300 ns/iter; 2.8296x vs baseline; 1.0581x over previous
import jax
import jax.numpy as jnp
from jax import lax
from jax.experimental import pallas as pl
from jax.experimental.pallas import tpu as pltpu

N_DEV = 32
B_LOC = 2
SQ = 128
SKV = 128
H_LOC = 4
DH = 64
D_MODEL = 512
HD_LOC = H_LOC * DH
ROWS = B_LOC * SQ
N_HOP = N_DEV // 2

_H = [0, 1, 2, 5, 6, 7, 4, 3,
      11, 12, 15, 14, 13, 10, 9,
      17, 18, 21, 22, 23, 20, 19,
      27, 28, 31, 30, 29, 26, 25,
      24, 16, 8]
_H_INV = [0] * N_DEV
for _p, _i in enumerate(_H):
    _H_INV[_i] = _p


def _body(x_ref, chunk_ref, k_ref, v_ref, meta_ref, out_ref,
          comm_r, comm_l, s_ref, w_ref, ctx_ref,
          send_ra, recv_ra, send_rb, recv_rb,
          send_la, recv_la, send_lb, recv_lb, cred_r, cred_l):
    right = meta_ref[0]
    left = meta_ref[1]

    barrier_sem = pltpu.get_barrier_semaphore()
    for nbr in (left, right):
        pl.semaphore_signal(barrier_sem, inc=1, device_id=(nbr,),
                            device_id_type=pl.DeviceIdType.MESH)
    pl.semaphore_wait(barrier_sem, 2)

    out_ref[:, :] = jnp.zeros((ROWS, D_MODEL), jnp.float32)
    comm_r[0, :, :] = chunk_ref[:, :]
    comm_l[0, :, :] = chunk_ref[:, :]

    def compute1(jj, comm, slot):
        wq_t = comm[pl.ds(slot, 1), pl.ds(0, HD_LOC), :][0]
        qf = (lax.dot_general(
            x_ref[:, :], wq_t, (((1,), (1,)), ((), ())),
            preferred_element_type=jnp.float32,
        ) * 0.125).astype(jnp.bfloat16)
        for p in range(B_LOC * H_LOC):
            b, hh = divmod(p, H_LOC)
            q = qf[b * SQ:(b + 1) * SQ, hh * DH:(hh + 1) * DH]
            idx = b * (N_DEV * H_LOC) + jj * H_LOC + hh
            k = k_ref[pl.ds(idx, 1), :, :][0]
            s_ref[p * SQ:(p + 1) * SQ, :] = lax.dot_general(
                q, k, (((1,), (1,)), ((), ())),
                preferred_element_type=jnp.float32,
            )
        sv = s_ref[:, :]
        m = jnp.max(sv, axis=-1, keepdims=True)
        e = jnp.exp(sv - m)
        w_ref[:, :] = (e / jnp.sum(e, axis=-1, keepdims=True)
                       ).astype(jnp.bfloat16)
        for p in range(B_LOC * H_LOC):
            b, hh = divmod(p, H_LOC)
            idx = b * (N_DEV * H_LOC) + jj * H_LOC + hh
            v = v_ref[pl.ds(idx, 1), :, :][0]
            c = lax.dot_general(
                w_ref[p * SQ:(p + 1) * SQ, :], v,
                (((1,), (0,)), ((), ())),
                preferred_element_type=jnp.float32,
            ).astype(jnp.bfloat16)
            ctx_ref[b * SQ:(b + 1) * SQ, hh * DH:(hh + 1) * DH] = c

    def compute2(comm, slot):
        wo = comm[pl.ds(slot, 1), pl.ds(HD_LOC, HD_LOC), :][0]
        part = lax.dot_general(
            ctx_ref[:, :], wo, (((1,), (0,)), ((), ())),
            preferred_element_type=jnp.float32,
        )
        out_ref[:, :] = out_ref[:, :] + part

    def half(buf, src_slot, dst_slot, row0, s_sem, r_sem, r_idx, dev):
        return pltpu.make_async_remote_copy(
            src_ref=buf.at[src_slot, pl.ds(row0, HD_LOC)],
            dst_ref=buf.at[dst_slot, pl.ds(row0, HD_LOC)],
            send_sem=s_sem.at[src_slot], recv_sem=r_sem.at[r_idx],
            device_id=(dev,), device_id_type=pl.DeviceIdType.MESH,
        )

    def hop(h, carry):
        slot = lax.rem(h, 4)
        nxt = lax.rem(h + 1, 4)

        @pl.when(h >= 3)
        def _():
            pl.semaphore_wait(cred_r, 1)

        @pl.when(jnp.logical_and(h >= 3, h <= N_HOP - 2))
        def _():
            pl.semaphore_wait(cred_l, 1)

        @pl.when(h >= 1)
        def _():
            half(comm_r, slot, nxt, 0, send_ra, recv_ra, slot,
                 right).wait_recv()
        fwd_ra = half(comm_r, slot, nxt, 0, send_ra, recv_ra, nxt, right)
        fwd_ra.start()

        @pl.when(h >= 1)
        def _():
            half(comm_r, slot, nxt, HD_LOC, send_rb, recv_rb, slot,
                 right).wait_recv()
        fwd_rb = half(comm_r, slot, nxt, HD_LOC, send_rb, recv_rb, nxt,
                      right)
        fwd_rb.start()

        @pl.when(h >= 1)
        def _():
            half(comm_l, slot, nxt, 0, send_la, recv_la, slot,
                 left).wait_recv()
        fwd_la = half(comm_l, slot, nxt, 0, send_la, recv_la, nxt, left)

        @pl.when(h <= N_HOP - 2)
        def _():
            fwd_la.start()

        @pl.when(h >= 1)
        def _():
            half(comm_l, slot, nxt, HD_LOC, send_lb, recv_lb, slot,
                 left).wait_recv()
        fwd_lb = half(comm_l, slot, nxt, HD_LOC, send_lb, recv_lb, nxt,
                      left)

        @pl.when(h <= N_HOP - 2)
        def _():
            fwd_lb.start()

        compute1(meta_ref[2 + h], comm_r, slot)
        compute2(comm_r, slot)

        @pl.when(h >= 1)
        def _():
            compute1(meta_ref[2 + N_HOP + 1 + h], comm_l, slot)
            compute2(comm_l, slot)

        fwd_ra.wait_send()
        fwd_rb.wait_send()

        @pl.when(h <= N_HOP - 2)
        def _():
            fwd_la.wait_send()
            fwd_lb.wait_send()

        @pl.when(h <= N_HOP - 4)
        def _():
            pl.semaphore_signal(cred_r, inc=1, device_id=(left,),
                                device_id_type=pl.DeviceIdType.MESH)

        @pl.when(h <= N_HOP - 5)
        def _():
            pl.semaphore_signal(cred_l, inc=1, device_id=(right,),
                                device_id_type=pl.DeviceIdType.MESH)
        return carry

    lax.fori_loop(0, N_HOP, hop, 0)
    half(comm_r, 0, 0, 0, send_ra, recv_ra, 0, right).wait_recv()
    compute1(meta_ref[2 + N_HOP], comm_r, 0)
    half(comm_r, 0, 0, HD_LOC, send_rb, recv_rb, 0, right).wait_recv()
    compute2(comm_r, 0)


def kernel(x, Wq, K_ext, V_ext, Wo):
    my = lax.axis_index("i")
    xf = x.reshape(ROWS, D_MODEL).astype(jnp.bfloat16)
    chunk = jnp.concatenate([Wq.T, Wo], axis=0).astype(jnp.bfloat16)

    b0 = my * B_LOC
    k_loc = lax.dynamic_slice(K_ext, (b0, 0, 0, 0), (B_LOC, SKV, 128, DH))
    v_loc = lax.dynamic_slice(V_ext, (b0, 0, 0, 0), (B_LOC, SKV, 128, DH))
    k_t = k_loc.transpose(0, 2, 1, 3).reshape(B_LOC * 128, SKV, DH)
    v_t = v_loc.transpose(0, 2, 1, 3).reshape(B_LOC * 128, SKV, DH)
    k_t = k_t.astype(jnp.bfloat16)
    v_t = v_t.astype(jnp.bfloat16)

    hj = jnp.array(_H, jnp.int32)
    p = jnp.array(_H_INV, jnp.int32)[my]
    right = hj[jnp.remainder(p + 1, N_DEV)]
    left = hj[jnp.remainder(p - 1, N_DEV)]
    orr = hj[jnp.remainder(p - jnp.arange(N_HOP + 1), N_DEV)]
    orl = hj[jnp.remainder(p + jnp.arange(N_HOP), N_DEV)]
    meta = jnp.concatenate(
        [right[None], left[None], orr, orl]).astype(jnp.int32)

    out = pl.pallas_call(
        _body,
        out_shape=jax.ShapeDtypeStruct((ROWS, D_MODEL), jnp.float32),
        in_specs=[pl.BlockSpec(memory_space=pltpu.VMEM)] * 4
        + [pl.BlockSpec(memory_space=pltpu.SMEM)],
        out_specs=pl.BlockSpec(memory_space=pltpu.VMEM),
        scratch_shapes=[
            pltpu.VMEM((4, 2 * HD_LOC, D_MODEL), jnp.bfloat16),
            pltpu.VMEM((4, 2 * HD_LOC, D_MODEL), jnp.bfloat16),
            pltpu.VMEM((B_LOC * H_LOC * SQ, SKV), jnp.float32),
            pltpu.VMEM((B_LOC * H_LOC * SQ, SKV), jnp.bfloat16),
            pltpu.VMEM((ROWS, HD_LOC), jnp.bfloat16),
            pltpu.SemaphoreType.DMA((4,)),
            pltpu.SemaphoreType.DMA((4,)),
            pltpu.SemaphoreType.DMA((4,)),
            pltpu.SemaphoreType.DMA((4,)),
            pltpu.SemaphoreType.DMA((4,)),
            pltpu.SemaphoreType.DMA((4,)),
            pltpu.SemaphoreType.DMA((4,)),
            pltpu.SemaphoreType.DMA((4,)),
            pltpu.SemaphoreType.REGULAR,
            pltpu.SemaphoreType.REGULAR,
        ],
        compiler_params=pltpu.CompilerParams(collective_id=0),
    )(xf, chunk, k_t, v_t, meta)
    return out.reshape(B_LOC, SQ, D_MODEL)
